# baseline (speedup 1.0000x reference)
import functools

import jax
import jax.numpy as jnp
from jax import lax
from jax.experimental import pallas as pl
from jax.experimental.pallas import tpu as pltpu
from jax.experimental.pallas import tpu_sc as plsc

H = 128


def _emb_body(w_ref, b_ref, attr_ref, out_ref):
    a = attr_ref[...]
    out_ref[...] = (
        a[:, 0 * H:1 * H] * w_ref[0]
        + a[:, 1 * H:2 * H] * w_ref[1]
        + a[:, 2 * H:3 * H] * w_ref[2]
        + a[:, 3 * H:4 * H] * w_ref[3]
        + b_ref[0]
    )


def _node_emb(node_attr2d, conv_w, conv_b):
    n = node_attr2d.shape[0]
    nb = 1000
    grid = (n // nb,)
    return pl.pallas_call(
        _emb_body,
        grid=grid,
        in_specs=[
            pl.BlockSpec(memory_space=pltpu.SMEM),
            pl.BlockSpec(memory_space=pltpu.SMEM),
            pl.BlockSpec((nb, 4 * H), lambda i: (i, 0)),
        ],
        out_specs=pl.BlockSpec((nb, H), lambda i: (i, 0)),
        out_shape=jax.ShapeDtypeStruct((n, H), jnp.float32),
    )(conv_w, conv_b, node_attr2d)


def _gather_src_tgt(node_emb, row, col):
    e = row.shape[0]
    nw = 32
    ch = 128
    n_chunks = e // ch
    base_trips = n_chunks // nw
    extra = n_chunks - base_trips * nw

    mesh = plsc.VectorSubcoreMesh(core_axis_name="c", subcore_axis_name="s")

    @functools.partial(
        pl.kernel,
        mesh=mesh,
        out_type=(
            jax.ShapeDtypeStruct((e, H), jnp.float32),
            jax.ShapeDtypeStruct((e, H), jnp.float32),
        ),
        scratch_types=[
            pltpu.VMEM((ch,), jnp.int32),
            pltpu.VMEM((ch, H), jnp.float32),
            pltpu.SemaphoreType.DMA,
        ],
    )
    def k(emb_hbm, row_hbm, col_hbm, src_out, tgt_out, idx_v, rows_v, sem):
        wid = lax.axis_index("s") * 2 + lax.axis_index("c")
        trips = base_trips + jnp.where(wid < extra, 1, 0)

        def one(idx_hbm, out_hbm, off):
            pltpu.sync_copy(idx_hbm.at[pl.ds(off, ch)], idx_v)
            pltpu.async_copy(emb_hbm.at[idx_v], rows_v, sem).wait()
            pltpu.sync_copy(rows_v, out_hbm.at[pl.ds(off, ch)])

        def body(i, carry):
            c = wid + nw * i
            off = c * ch
            one(row_hbm, src_out, off)
            one(col_hbm, tgt_out, off)
            return carry

        lax.fori_loop(0, trips, body, 0)

    return k(node_emb, row, col)


def _mlp_body(src_ref, tgt_ref, edge_ref, w1s_ref, w1t_ref, w1e_ref,
              b1_ref, w2_ref, b2_ref, w3_ref, b3_ref, out_ref):
    f32 = jnp.float32
    bf16 = jnp.bfloat16
    h = (
        jnp.dot(src_ref[...].astype(bf16), w1s_ref[...],
                preferred_element_type=f32)
        + jnp.dot(tgt_ref[...].astype(bf16), w1t_ref[...],
                  preferred_element_type=f32)
        + jnp.dot(edge_ref[...].astype(bf16), w1e_ref[...],
                  preferred_element_type=f32)
        + b1_ref[...]
    )
    h = jnp.maximum(h, 0.0).astype(bf16)
    h = jnp.maximum(
        jnp.dot(h, w2_ref[...], preferred_element_type=f32) + b2_ref[...], 0.0
    ).astype(bf16)
    out_ref[...] = (
        jnp.dot(h, w3_ref[...], preferred_element_type=f32) + b3_ref[...])


def _edge_mlp(src, tgt, edge_input, w1t, b1, w2t, b2, w3t, b3):
    e = src.shape[0]
    eb = 2000
    grid = (e // eb,)
    d1 = w1t.shape[1]
    d2 = w2t.shape[1]
    d3 = w3t.shape[1]
    blk = lambda r, c: pl.BlockSpec((r, c), lambda i: (i, 0))
    full = lambda r, c: pl.BlockSpec((r, c), lambda i: (0, 0))
    return pl.pallas_call(
        _mlp_body,
        grid=grid,
        in_specs=[
            blk(eb, H), blk(eb, H), blk(eb, H),
            full(H, d1), full(H, d1), full(H, d1), full(1, d1),
            full(d1, d2), full(1, d2),
            full(d2, d3), full(1, d3),
        ],
        out_specs=pl.BlockSpec((eb, d3), lambda i: (i, 0)),
        out_shape=jax.ShapeDtypeStruct((e, d3), jnp.float32),
    )(src, tgt, edge_input,
      w1t[0 * H:1 * H], w1t[1 * H:2 * H], w1t[2 * H:3 * H], b1[None, :],
      w2t, b2[None, :], w3t, b3[None, :])


def _bf16(x):
    return x.astype(jnp.bfloat16)


def kernel(node_attr, edge_input, edge_index, conv_w, conv_b,
           W1, b1, W2, b2, W3, b3):
    n = node_attr.shape[0]
    node_attr2d = node_attr.reshape(n, 4 * H)
    emb = _node_emb(node_attr2d, conv_w, conv_b)
    row = edge_index[0].astype(jnp.int32)
    col = edge_index[1].astype(jnp.int32)
    src, tgt = _gather_src_tgt(emb, row, col)
    return _edge_mlp(src, tgt, edge_input,
                     _bf16(W1.T), b1, _bf16(W2.T), b2, _bf16(W3.T), b3)


# speedup vs baseline: 1.0890x; 1.0890x over previous
import functools

import jax
import jax.numpy as jnp
from jax import lax
from jax.experimental import pallas as pl
from jax.experimental.pallas import tpu as pltpu
from jax.experimental.pallas import tpu_sc as plsc

H = 128


def _emb_body(w_ref, b_ref, attr_ref, out_ref):
    a = attr_ref[...]
    out_ref[...] = (
        a[:, 0 * H:1 * H] * w_ref[0]
        + a[:, 1 * H:2 * H] * w_ref[1]
        + a[:, 2 * H:3 * H] * w_ref[2]
        + a[:, 3 * H:4 * H] * w_ref[3]
        + b_ref[0]
    )


def _node_emb(node_attr2d, conv_w, conv_b):
    n = node_attr2d.shape[0]
    nb = 1000
    grid = (n // nb,)
    return pl.pallas_call(
        _emb_body,
        grid=grid,
        in_specs=[
            pl.BlockSpec(memory_space=pltpu.SMEM),
            pl.BlockSpec(memory_space=pltpu.SMEM),
            pl.BlockSpec((nb, 4 * H), lambda i: (i, 0)),
        ],
        out_specs=pl.BlockSpec((nb, H), lambda i: (i, 0)),
        out_shape=jax.ShapeDtypeStruct((n, H), jnp.float32),
    )(conv_w, conv_b, node_attr2d)


def _gather_src_tgt(node_emb, row, col):
    e = row.shape[0]
    nw = 32
    ch = 128
    n_chunks = e // ch
    base_trips = n_chunks // nw
    extra = n_chunks - base_trips * nw

    mesh = plsc.VectorSubcoreMesh(core_axis_name="c", subcore_axis_name="s")

    @functools.partial(
        pl.kernel,
        mesh=mesh,
        out_type=jax.ShapeDtypeStruct((e, 2 * H), jnp.float32),
        scratch_types=[
            pltpu.VMEM((ch,), jnp.int32),
            pltpu.VMEM((ch, H), jnp.float32),
            pltpu.SemaphoreType.DMA,
        ],
    )
    def k(emb_hbm, row_hbm, col_hbm, cat_out, idx_v, rows_v, sem):
        wid = lax.axis_index("s") * 2 + lax.axis_index("c")
        trips = base_trips + jnp.where(wid < extra, 1, 0)

        def one(idx_hbm, off, colo):
            pltpu.sync_copy(idx_hbm.at[pl.ds(off, ch)], idx_v)
            pltpu.async_copy(emb_hbm.at[idx_v], rows_v, sem).wait()
            pltpu.sync_copy(rows_v, cat_out.at[pl.ds(off, ch), pl.ds(colo, H)])

        def body(i, carry):
            c = wid + nw * i
            off = c * ch
            one(row_hbm, off, 0)
            one(col_hbm, off, H)
            return carry

        lax.fori_loop(0, trips, body, 0)

    return k(node_emb, row, col)


def _mlp_body(cat_ref, edge_ref, w1st_ref, w1e_ref,
              b1_ref, w2_ref, b2_ref, w3_ref, b3_ref, out_ref):
    f32 = jnp.float32
    h = (
        jnp.dot(cat_ref[...], w1st_ref[...], preferred_element_type=f32)
        + jnp.dot(edge_ref[...], w1e_ref[...], preferred_element_type=f32)
        + b1_ref[...]
    )
    h = jnp.maximum(h, 0.0)
    h = jnp.maximum(
        jnp.dot(h, w2_ref[...], preferred_element_type=f32) + b2_ref[...], 0.0)
    out_ref[...] = (
        jnp.dot(h, w3_ref[...], preferred_element_type=f32) + b3_ref[...])


def _edge_mlp(cat, edge_input, w1t, b1, w2t, b2, w3t, b3):
    e = cat.shape[0]
    eb = 2000
    grid = (e // eb,)
    d1 = w1t.shape[1]
    d2 = w2t.shape[1]
    d3 = w3t.shape[1]
    blk = lambda r, c: pl.BlockSpec((r, c), lambda i: (i, 0))
    full = lambda r, c: pl.BlockSpec((r, c), lambda i: (0, 0))
    return pl.pallas_call(
        _mlp_body,
        grid=grid,
        in_specs=[
            blk(eb, 2 * H), blk(eb, H),
            full(2 * H, d1), full(H, d1), full(1, d1),
            full(d1, d2), full(1, d2),
            full(d2, d3), full(1, d3),
        ],
        out_specs=pl.BlockSpec((eb, d3), lambda i: (i, 0)),
        out_shape=jax.ShapeDtypeStruct((e, d3), jnp.float32),
    )(cat, edge_input,
      w1t[0:2 * H], w1t[2 * H:3 * H], b1[None, :],
      w2t, b2[None, :], w3t, b3[None, :])


def kernel(node_attr, edge_input, edge_index, conv_w, conv_b,
           W1, b1, W2, b2, W3, b3):
    n = node_attr.shape[0]
    node_attr2d = node_attr.reshape(n, 4 * H)
    emb = _node_emb(node_attr2d, conv_w, conv_b)
    row = edge_index[0].astype(jnp.int32)
    col = edge_index[1].astype(jnp.int32)
    cat = _gather_src_tgt(emb, row, col)
    return _edge_mlp(cat, edge_input, W1.T, b1, W2.T, b2, W3.T, b3)


# speedup vs baseline: 1.3791x; 1.2665x over previous
import functools

import jax
import jax.numpy as jnp
from jax import lax
from jax.experimental import pallas as pl
from jax.experimental.pallas import tpu as pltpu
from jax.experimental.pallas import tpu_sc as plsc

H = 128


def _emb_body(w_ref, b_ref, attr_ref, out_ref):
    a = attr_ref[...]
    out_ref[...] = (
        a[:, 0 * H:1 * H] * w_ref[0]
        + a[:, 1 * H:2 * H] * w_ref[1]
        + a[:, 2 * H:3 * H] * w_ref[2]
        + a[:, 3 * H:4 * H] * w_ref[3]
        + b_ref[0]
    )


def _node_emb(node_attr2d, conv_w, conv_b):
    n = node_attr2d.shape[0]
    nb = 1000
    grid = (n // nb,)
    return pl.pallas_call(
        _emb_body,
        grid=grid,
        in_specs=[
            pl.BlockSpec(memory_space=pltpu.SMEM),
            pl.BlockSpec(memory_space=pltpu.SMEM),
            pl.BlockSpec((nb, 4 * H), lambda i: (i, 0)),
        ],
        out_specs=pl.BlockSpec((nb, H), lambda i: (i, 0)),
        out_shape=jax.ShapeDtypeStruct((n, H), jnp.float32),
    )(conv_w, conv_b, node_attr2d)


def _gather_src_tgt(node_emb, row, col):
    e = row.shape[0]
    nw = 32
    ch = 128
    n_chunks = e // ch
    base_trips = n_chunks // nw
    extra = n_chunks - base_trips * nw

    mesh = plsc.VectorSubcoreMesh(core_axis_name="c", subcore_axis_name="s")

    @functools.partial(
        pl.kernel,
        mesh=mesh,
        out_type=jax.ShapeDtypeStruct((e, 2 * H), jnp.float32),
        scratch_types=[
            pltpu.VMEM((ch,), jnp.int32),
            pltpu.VMEM((ch, H), jnp.float32),
            pltpu.SemaphoreType.DMA,
        ],
    )
    def k(emb_hbm, row_hbm, col_hbm, cat_out, idx_v, rows_v, sem):
        wid = lax.axis_index("s") * 2 + lax.axis_index("c")
        trips = base_trips + jnp.where(wid < extra, 1, 0)

        def one(idx_hbm, off, colo):
            pltpu.sync_copy(idx_hbm.at[pl.ds(off, ch)], idx_v)
            pltpu.async_copy(emb_hbm.at[idx_v], rows_v, sem).wait()
            pltpu.sync_copy(rows_v, cat_out.at[pl.ds(off, ch), pl.ds(colo, H)])

        def body(i, carry):
            c = wid + nw * i
            off = c * ch
            one(row_hbm, off, 0)
            one(col_hbm, off, H)
            return carry

        lax.fori_loop(0, trips, body, 0)

    return k(node_emb, row, col)


def _mlp_body(cat_ref, edge_ref, w1st_ref, w1e_ref,
              b1_ref, w2_ref, b2_ref, w3_ref, b3_ref, out_ref):
    f32 = jnp.float32
    h = (
        jnp.dot(cat_ref[...], w1st_ref[...], preferred_element_type=f32)
        + jnp.dot(edge_ref[...], w1e_ref[...], preferred_element_type=f32)
        + b1_ref[...]
    )
    h = jnp.maximum(h, 0.0)
    h = jnp.maximum(
        jnp.dot(h, w2_ref[...], preferred_element_type=f32) + b2_ref[...], 0.0)
    out_ref[...] = (
        jnp.dot(h, w3_ref[...], preferred_element_type=f32) + b3_ref[...])


def _mlp_body_buf(cat_ref, edge_ref, w1st_ref, w1e_ref, b1_ref, w2_ref,
                  b2_ref, w3_ref, b3_ref, buf_ref, out_ref):
    del buf_ref
    _mlp_body(cat_ref, edge_ref, w1st_ref, w1e_ref, b1_ref, w2_ref,
              b2_ref, w3_ref, b3_ref, out_ref)


def _edge_mlp_chunk(cat, edge_input, w1t, b1, w2t, b2, w3t, b3, buf, base_blk):
    ec = cat.shape[0]
    e = edge_input.shape[0]
    eb = 2000
    nblk = ec // eb
    d1 = w1t.shape[1]
    d2 = w2t.shape[1]
    d3 = w3t.shape[1]
    loc = lambda r, c: pl.BlockSpec((r, c), lambda i: (i, 0))
    glb = lambda r, c: pl.BlockSpec((r, c), lambda i: (i + base_blk, 0))
    full = lambda r, c: pl.BlockSpec((r, c), lambda i: (0, 0))
    in_specs = [
        loc(eb, 2 * H), glb(eb, H),
        full(2 * H, d1), full(H, d1), full(1, d1),
        full(d1, d2), full(1, d2),
        full(d2, d3), full(1, d3),
    ]
    args = [cat, edge_input,
            w1t[0:2 * H], w1t[2 * H:3 * H], b1[None, :],
            w2t, b2[None, :], w3t, b3[None, :]]
    kwargs = {}
    body = _mlp_body
    if buf is not None:
        in_specs.append(pl.BlockSpec(memory_space=pl.ANY))
        args.append(buf)
        body = _mlp_body_buf
        kwargs["input_output_aliases"] = {9: 0}
    return pl.pallas_call(
        body,
        grid=(nblk,),
        in_specs=in_specs,
        out_specs=pl.BlockSpec((eb, d3), lambda i: (i + base_blk, 0)),
        out_shape=jax.ShapeDtypeStruct((e, d3), jnp.float32),
        **kwargs,
    )(*args)


def kernel(node_attr, edge_input, edge_index, conv_w, conv_b,
           W1, b1, W2, b2, W3, b3):
    n = node_attr.shape[0]
    node_attr2d = node_attr.reshape(n, 4 * H)
    emb = _node_emb(node_attr2d, conv_w, conv_b)
    row = edge_index[0].astype(jnp.int32)
    col = edge_index[1].astype(jnp.int32)
    nch = 4
    e = row.shape[0]
    ec = e // nch
    w1t = W1.T
    cats = [
        _gather_src_tgt(emb,
                        lax.slice(row, (c * ec,), ((c + 1) * ec,)),
                        lax.slice(col, (c * ec,), ((c + 1) * ec,)))
        for c in range(nch)
    ]
    out = None
    eb = 2000
    for c in range(nch):
        out = _edge_mlp_chunk(cats[c], edge_input, w1t, b1, W2.T, b2, W3.T,
                              b3, out, c * (ec // eb))
    return out


# speedup vs baseline: 1.5657x; 1.1353x over previous
import functools

import jax
import jax.numpy as jnp
from jax import lax
from jax.experimental import pallas as pl
from jax.experimental.pallas import tpu as pltpu
from jax.experimental.pallas import tpu_sc as plsc

H = 128


def _emb_body(w_ref, b_ref, attr_ref, out_ref):
    a = attr_ref[...]
    out_ref[...] = (
        a[:, 0 * H:1 * H] * w_ref[0]
        + a[:, 1 * H:2 * H] * w_ref[1]
        + a[:, 2 * H:3 * H] * w_ref[2]
        + a[:, 3 * H:4 * H] * w_ref[3]
        + b_ref[0]
    )


def _node_emb(node_attr2d, conv_w, conv_b):
    n = node_attr2d.shape[0]
    nb = 1000
    grid = (n // nb,)
    return pl.pallas_call(
        _emb_body,
        grid=grid,
        in_specs=[
            pl.BlockSpec(memory_space=pltpu.SMEM),
            pl.BlockSpec(memory_space=pltpu.SMEM),
            pl.BlockSpec((nb, 4 * H), lambda i: (i, 0)),
        ],
        out_specs=pl.BlockSpec((nb, H), lambda i: (i, 0)),
        out_shape=jax.ShapeDtypeStruct((n, H), jnp.float32),
    )(conv_w, conv_b, node_attr2d)


def _gather_src_tgt(node_emb, row, col):
    e = row.shape[0]
    nw = 32
    ch = 128
    n_chunks = e // ch
    base_trips = n_chunks // nw
    extra = n_chunks - base_trips * nw

    mesh = plsc.VectorSubcoreMesh(core_axis_name="c", subcore_axis_name="s")

    @functools.partial(
        pl.kernel,
        mesh=mesh,
        out_type=jax.ShapeDtypeStruct((e, 2 * H), jnp.float32),
        scratch_types=[
            pltpu.VMEM((ch,), jnp.int32),
            pltpu.VMEM((ch,), jnp.int32),
            pltpu.VMEM((ch, H), jnp.float32),
            pltpu.VMEM((ch, H), jnp.float32),
            pltpu.SemaphoreType.DMA,
            pltpu.SemaphoreType.DMA,
            pltpu.SemaphoreType.DMA,
            pltpu.SemaphoreType.DMA,
            pltpu.SemaphoreType.DMA,
            pltpu.SemaphoreType.DMA,
        ],
    )
    def k(emb_hbm, row_hbm, col_hbm, cat_out, idxr_v, idxc_v, rows_r, rows_c,
          sem_ir, sem_ic, sem_gr, sem_gc, sem_sr, sem_sc):
        wid = lax.axis_index("s") * 2 + lax.axis_index("c")
        trips = base_trips + jnp.where(wid < extra, 1, 0)

        def store_slot(off, colo):
            return cat_out.at[pl.ds(off, ch), pl.ds(colo, H)]

        def body(i, carry):
            off = (wid + nw * i) * ch
            hir = pltpu.async_copy(row_hbm.at[pl.ds(off, ch)], idxr_v, sem_ir)
            hic = pltpu.async_copy(col_hbm.at[pl.ds(off, ch)], idxc_v, sem_ic)

            @pl.when(i > 0)
            def _():
                pltpu.make_async_copy(rows_r, store_slot(off, 0), sem_sr).wait()
                pltpu.make_async_copy(rows_c, store_slot(off, H), sem_sc).wait()

            hir.wait()
            gr = pltpu.async_copy(emb_hbm.at[idxr_v], rows_r, sem_gr)
            hic.wait()
            gc = pltpu.async_copy(emb_hbm.at[idxc_v], rows_c, sem_gc)
            gr.wait()
            pltpu.async_copy(rows_r, store_slot(off, 0), sem_sr)
            gc.wait()
            pltpu.async_copy(rows_c, store_slot(off, H), sem_sc)
            return carry

        lax.fori_loop(0, trips, body, 0)
        pltpu.make_async_copy(rows_r, store_slot(0, 0), sem_sr).wait()
        pltpu.make_async_copy(rows_c, store_slot(0, H), sem_sc).wait()

    return k(node_emb, row, col)


def _mlp_body(cat_ref, edge_ref, w1st_ref, w1e_ref,
              b1_ref, w2_ref, b2_ref, w3_ref, b3_ref, out_ref):
    f32 = jnp.float32
    h = (
        jnp.dot(cat_ref[...], w1st_ref[...], preferred_element_type=f32)
        + jnp.dot(edge_ref[...], w1e_ref[...], preferred_element_type=f32)
        + b1_ref[...]
    )
    h = jnp.maximum(h, 0.0)
    h = jnp.maximum(
        jnp.dot(h, w2_ref[...], preferred_element_type=f32) + b2_ref[...], 0.0)
    out_ref[...] = (
        jnp.dot(h, w3_ref[...], preferred_element_type=f32) + b3_ref[...])


def _mlp_body_buf(cat_ref, edge_ref, w1st_ref, w1e_ref, b1_ref, w2_ref,
                  b2_ref, w3_ref, b3_ref, buf_ref, out_ref):
    del buf_ref
    _mlp_body(cat_ref, edge_ref, w1st_ref, w1e_ref, b1_ref, w2_ref,
              b2_ref, w3_ref, b3_ref, out_ref)


def _edge_mlp_chunk(cat, edge_input, w1t, b1, w2t, b2, w3t, b3, buf, base_blk):
    ec = cat.shape[0]
    e = edge_input.shape[0]
    eb = 2000
    nblk = ec // eb
    d1 = w1t.shape[1]
    d2 = w2t.shape[1]
    d3 = w3t.shape[1]
    loc = lambda r, c: pl.BlockSpec((r, c), lambda i: (i, 0))
    glb = lambda r, c: pl.BlockSpec((r, c), lambda i: (i + base_blk, 0))
    full = lambda r, c: pl.BlockSpec((r, c), lambda i: (0, 0))
    in_specs = [
        loc(eb, 2 * H), glb(eb, H),
        full(2 * H, d1), full(H, d1), full(1, d1),
        full(d1, d2), full(1, d2),
        full(d2, d3), full(1, d3),
    ]
    args = [cat, edge_input,
            w1t[0:2 * H], w1t[2 * H:3 * H], b1[None, :],
            w2t, b2[None, :], w3t, b3[None, :]]
    kwargs = {}
    body = _mlp_body
    if buf is not None:
        in_specs.append(pl.BlockSpec(memory_space=pl.ANY))
        args.append(buf)
        body = _mlp_body_buf
        kwargs["input_output_aliases"] = {9: 0}
    return pl.pallas_call(
        body,
        grid=(nblk,),
        in_specs=in_specs,
        out_specs=pl.BlockSpec((eb, d3), lambda i: (i + base_blk, 0)),
        out_shape=jax.ShapeDtypeStruct((e, d3), jnp.float32),
        **kwargs,
    )(*args)


def kernel(node_attr, edge_input, edge_index, conv_w, conv_b,
           W1, b1, W2, b2, W3, b3):
    n = node_attr.shape[0]
    node_attr2d = node_attr.reshape(n, 4 * H)
    emb = _node_emb(node_attr2d, conv_w, conv_b)
    row = edge_index[0].astype(jnp.int32)
    col = edge_index[1].astype(jnp.int32)
    nch = 4
    e = row.shape[0]
    ec = e // nch
    w1t = W1.T
    cats = [
        _gather_src_tgt(emb,
                        lax.slice(row, (c * ec,), ((c + 1) * ec,)),
                        lax.slice(col, (c * ec,), ((c + 1) * ec,)))
        for c in range(nch)
    ]
    out = None
    eb = 2000
    for c in range(nch):
        out = _edge_mlp_chunk(cats[c], edge_input, w1t, b1, W2.T, b2, W3.T,
                              b3, out, c * (ec // eb))
    return out


# speedup vs baseline: 1.5736x; 1.0051x over previous
import functools

import jax
import jax.numpy as jnp
from jax import lax
from jax.experimental import pallas as pl
from jax.experimental.pallas import tpu as pltpu
from jax.experimental.pallas import tpu_sc as plsc

H = 128


def _emb_body(w_ref, b_ref, attr_ref, out_ref):
    a = attr_ref[...]
    out_ref[...] = (
        a[:, 0 * H:1 * H] * w_ref[0]
        + a[:, 1 * H:2 * H] * w_ref[1]
        + a[:, 2 * H:3 * H] * w_ref[2]
        + a[:, 3 * H:4 * H] * w_ref[3]
        + b_ref[0]
    )


def _node_emb(node_attr2d, conv_w, conv_b):
    n = node_attr2d.shape[0]
    nb = 1000
    grid = (n // nb,)
    return pl.pallas_call(
        _emb_body,
        grid=grid,
        in_specs=[
            pl.BlockSpec(memory_space=pltpu.SMEM),
            pl.BlockSpec(memory_space=pltpu.SMEM),
            pl.BlockSpec((nb, 4 * H), lambda i: (i, 0)),
        ],
        out_specs=pl.BlockSpec((nb, H), lambda i: (i, 0)),
        out_shape=jax.ShapeDtypeStruct((n, H), jnp.float32),
    )(conv_w, conv_b, node_attr2d)


def _gather_src_tgt(node_emb, row, col):
    e = row.shape[0]
    nw = 32
    ch = 128
    n_chunks = e // ch
    base_trips = n_chunks // nw
    extra = n_chunks - base_trips * nw

    mesh = plsc.VectorSubcoreMesh(core_axis_name="c", subcore_axis_name="s")

    @functools.partial(
        pl.kernel,
        mesh=mesh,
        out_type=jax.ShapeDtypeStruct((e, 2 * H), jnp.float32),
        scratch_types=[
            pltpu.VMEM((ch,), jnp.int32),
            pltpu.VMEM((ch,), jnp.int32),
            pltpu.VMEM((2, ch, H), jnp.float32),
            pltpu.VMEM((2, ch, H), jnp.float32),
            pltpu.SemaphoreType.DMA,
            pltpu.SemaphoreType.DMA,
            pltpu.SemaphoreType.DMA,
            pltpu.SemaphoreType.DMA,
            pltpu.SemaphoreType.DMA,
            pltpu.SemaphoreType.DMA,
            pltpu.SemaphoreType.DMA,
            pltpu.SemaphoreType.DMA,
        ],
    )
    def k(emb_hbm, row_hbm, col_hbm, cat_out, idxr_v, idxc_v, rows_r, rows_c,
          sem_ir, sem_ic, sem_gr, sem_gc, sem_sr0, sem_sr1, sem_sc0, sem_sc1):
        wid = lax.axis_index("s") * 2 + lax.axis_index("c")
        trips = base_trips + jnp.where(wid < extra, 1, 0)
        sem_sr = (sem_sr0, sem_sr1)
        sem_sc = (sem_sc0, sem_sc1)

        def store_slot(off, colo):
            return cat_out.at[pl.ds(off, ch), pl.ds(colo, H)]

        def trip(t, b):
            off = (wid + nw * t) * ch
            hir = pltpu.async_copy(row_hbm.at[pl.ds(off, ch)], idxr_v, sem_ir)
            hic = pltpu.async_copy(col_hbm.at[pl.ds(off, ch)], idxc_v, sem_ic)

            @pl.when(t >= 2)
            def _():
                pltpu.make_async_copy(
                    rows_r.at[b], store_slot(off, 0), sem_sr[b]).wait()
                pltpu.make_async_copy(
                    rows_c.at[b], store_slot(off, H), sem_sc[b]).wait()

            hir.wait()
            gr = pltpu.async_copy(emb_hbm.at[idxr_v], rows_r.at[b], sem_gr)
            hic.wait()
            gc = pltpu.async_copy(emb_hbm.at[idxc_v], rows_c.at[b], sem_gc)
            gr.wait()
            pltpu.async_copy(rows_r.at[b], store_slot(off, 0), sem_sr[b])
            gc.wait()
            pltpu.async_copy(rows_c.at[b], store_slot(off, H), sem_sc[b])

        def body(j, carry):
            for b in range(2):
                t = 2 * j + b

                @pl.when(t < trips)
                def _():
                    trip(t, b)
            return carry

        lax.fori_loop(0, (base_trips + 2) // 2, body, 0)
        for b in range(2):
            pltpu.make_async_copy(rows_r.at[b], store_slot(0, 0),
                                  sem_sr[b]).wait()
            pltpu.make_async_copy(rows_c.at[b], store_slot(0, H),
                                  sem_sc[b]).wait()

    return k(node_emb, row, col)


def _mlp_body(cat_ref, edge_ref, w1st_ref, w1e_ref,
              b1_ref, w2_ref, b2_ref, w3_ref, b3_ref, out_ref):
    f32 = jnp.float32
    h = (
        jnp.dot(cat_ref[...], w1st_ref[...], preferred_element_type=f32)
        + jnp.dot(edge_ref[...], w1e_ref[...], preferred_element_type=f32)
        + b1_ref[...]
    )
    h = jnp.maximum(h, 0.0)
    h = jnp.maximum(
        jnp.dot(h, w2_ref[...], preferred_element_type=f32) + b2_ref[...], 0.0)
    out_ref[...] = (
        jnp.dot(h, w3_ref[...], preferred_element_type=f32) + b3_ref[...])


def _mlp_body_buf(cat_ref, edge_ref, w1st_ref, w1e_ref, b1_ref, w2_ref,
                  b2_ref, w3_ref, b3_ref, buf_ref, out_ref):
    del buf_ref
    _mlp_body(cat_ref, edge_ref, w1st_ref, w1e_ref, b1_ref, w2_ref,
              b2_ref, w3_ref, b3_ref, out_ref)


def _edge_mlp_chunk(cat, edge_input, w1t, b1, w2t, b2, w3t, b3, buf, base_blk):
    ec = cat.shape[0]
    e = edge_input.shape[0]
    eb = 2000
    nblk = ec // eb
    d1 = w1t.shape[1]
    d2 = w2t.shape[1]
    d3 = w3t.shape[1]
    loc = lambda r, c: pl.BlockSpec((r, c), lambda i: (i, 0))
    glb = lambda r, c: pl.BlockSpec((r, c), lambda i: (i + base_blk, 0))
    full = lambda r, c: pl.BlockSpec((r, c), lambda i: (0, 0))
    in_specs = [
        loc(eb, 2 * H), glb(eb, H),
        full(2 * H, d1), full(H, d1), full(1, d1),
        full(d1, d2), full(1, d2),
        full(d2, d3), full(1, d3),
    ]
    args = [cat, edge_input,
            w1t[0:2 * H], w1t[2 * H:3 * H], b1[None, :],
            w2t, b2[None, :], w3t, b3[None, :]]
    kwargs = {}
    body = _mlp_body
    if buf is not None:
        in_specs.append(pl.BlockSpec(memory_space=pl.ANY))
        args.append(buf)
        body = _mlp_body_buf
        kwargs["input_output_aliases"] = {9: 0}
    return pl.pallas_call(
        body,
        grid=(nblk,),
        in_specs=in_specs,
        out_specs=pl.BlockSpec((eb, d3), lambda i: (i + base_blk, 0)),
        out_shape=jax.ShapeDtypeStruct((e, d3), jnp.float32),
        **kwargs,
    )(*args)


def kernel(node_attr, edge_input, edge_index, conv_w, conv_b,
           W1, b1, W2, b2, W3, b3):
    n = node_attr.shape[0]
    node_attr2d = node_attr.reshape(n, 4 * H)
    emb = _node_emb(node_attr2d, conv_w, conv_b)
    row = edge_index[0].astype(jnp.int32)
    col = edge_index[1].astype(jnp.int32)
    nch = 4
    e = row.shape[0]
    ec = e // nch
    w1t = W1.T
    cats = [
        _gather_src_tgt(emb,
                        lax.slice(row, (c * ec,), ((c + 1) * ec,)),
                        lax.slice(col, (c * ec,), ((c + 1) * ec,)))
        for c in range(nch)
    ]
    out = None
    eb = 2000
    for c in range(nch):
        out = _edge_mlp_chunk(cats[c], edge_input, w1t, b1, W2.T, b2, W3.T,
                              b3, out, c * (ec // eb))
    return out


# speedup vs baseline: 1.5824x; 1.0056x over previous
import functools

import jax
import jax.numpy as jnp
from jax import lax
from jax.experimental import pallas as pl
from jax.experimental.pallas import tpu as pltpu
from jax.experimental.pallas import tpu_sc as plsc

H = 128


def _emb_body(w_ref, b_ref, attr_ref, out_ref):
    a = attr_ref[...]
    out_ref[...] = (
        a[:, 0 * H:1 * H] * w_ref[0]
        + a[:, 1 * H:2 * H] * w_ref[1]
        + a[:, 2 * H:3 * H] * w_ref[2]
        + a[:, 3 * H:4 * H] * w_ref[3]
        + b_ref[0]
    )


def _node_emb(node_attr2d, conv_w, conv_b):
    n = node_attr2d.shape[0]
    nb = 1000
    grid = (n // nb,)
    return pl.pallas_call(
        _emb_body,
        grid=grid,
        in_specs=[
            pl.BlockSpec(memory_space=pltpu.SMEM),
            pl.BlockSpec(memory_space=pltpu.SMEM),
            pl.BlockSpec((nb, 4 * H), lambda i: (i, 0)),
        ],
        out_specs=pl.BlockSpec((nb, H), lambda i: (i, 0)),
        out_shape=jax.ShapeDtypeStruct((n, H), jnp.float32),
    )(conv_w, conv_b, node_attr2d)


def _gather_src_tgt(node_emb, row, col, base_e, ec):
    e = row.shape[0]
    nw = 32
    ch = 128
    n_chunks = ec // ch
    base_trips = n_chunks // nw
    extra = n_chunks - base_trips * nw

    mesh = plsc.VectorSubcoreMesh(core_axis_name="c", subcore_axis_name="s")

    @functools.partial(
        pl.kernel,
        mesh=mesh,
        out_type=jax.ShapeDtypeStruct((ec, 2 * H), jnp.float32),
        scratch_types=[
            pltpu.VMEM((ch,), jnp.int32),
            pltpu.VMEM((ch,), jnp.int32),
            pltpu.VMEM((2, ch, H), jnp.float32),
            pltpu.VMEM((2, ch, H), jnp.float32),
            pltpu.SemaphoreType.DMA,
            pltpu.SemaphoreType.DMA,
            pltpu.SemaphoreType.DMA,
            pltpu.SemaphoreType.DMA,
            pltpu.SemaphoreType.DMA,
            pltpu.SemaphoreType.DMA,
            pltpu.SemaphoreType.DMA,
            pltpu.SemaphoreType.DMA,
        ],
    )
    def k(emb_hbm, row_hbm, col_hbm, cat_out, idxr_v, idxc_v, rows_r, rows_c,
          sem_ir, sem_ic, sem_gr, sem_gc, sem_sr0, sem_sr1, sem_sc0, sem_sc1):
        wid = lax.axis_index("s") * 2 + lax.axis_index("c")
        trips = base_trips + jnp.where(wid < extra, 1, 0)
        sem_sr = (sem_sr0, sem_sr1)
        sem_sc = (sem_sc0, sem_sc1)

        def store_slot(off, colo):
            return cat_out.at[pl.ds(off, ch), pl.ds(colo, H)]

        def trip(t, b):
            off = (wid + nw * t) * ch
            hir = pltpu.async_copy(row_hbm.at[pl.ds(base_e + off, ch)],
                                   idxr_v, sem_ir)
            hic = pltpu.async_copy(col_hbm.at[pl.ds(base_e + off, ch)],
                                   idxc_v, sem_ic)

            @pl.when(t >= 2)
            def _():
                pltpu.make_async_copy(
                    rows_r.at[b], store_slot(off, 0), sem_sr[b]).wait()
                pltpu.make_async_copy(
                    rows_c.at[b], store_slot(off, H), sem_sc[b]).wait()

            hir.wait()
            gr = pltpu.async_copy(emb_hbm.at[idxr_v], rows_r.at[b], sem_gr)
            hic.wait()
            gc = pltpu.async_copy(emb_hbm.at[idxc_v], rows_c.at[b], sem_gc)
            gr.wait()
            pltpu.async_copy(rows_r.at[b], store_slot(off, 0), sem_sr[b])
            gc.wait()
            pltpu.async_copy(rows_c.at[b], store_slot(off, H), sem_sc[b])

        def body(j, carry):
            for b in range(2):
                t = 2 * j + b

                @pl.when(t < trips)
                def _():
                    trip(t, b)
            return carry

        lax.fori_loop(0, (base_trips + 2) // 2, body, 0)
        for b in range(2):
            pltpu.make_async_copy(rows_r.at[b], store_slot(0, 0),
                                  sem_sr[b]).wait()
            pltpu.make_async_copy(rows_c.at[b], store_slot(0, H),
                                  sem_sc[b]).wait()

    return k(node_emb, row, col)


def _mlp_body(cat_ref, edge_ref, w1st_ref, w1e_ref,
              b1_ref, w2_ref, b2_ref, w3_ref, b3_ref, out_ref):
    f32 = jnp.float32
    h = (
        jnp.dot(cat_ref[...], w1st_ref[...], preferred_element_type=f32)
        + jnp.dot(edge_ref[...], w1e_ref[...], preferred_element_type=f32)
        + b1_ref[...]
    )
    h = jnp.maximum(h, 0.0)
    h = jnp.maximum(
        jnp.dot(h, w2_ref[...], preferred_element_type=f32) + b2_ref[...], 0.0)
    out_ref[...] = (
        jnp.dot(h, w3_ref[...], preferred_element_type=f32) + b3_ref[...])


def _mlp_body_buf(cat_ref, edge_ref, w1st_ref, w1e_ref, b1_ref, w2_ref,
                  b2_ref, w3_ref, b3_ref, buf_ref, out_ref):
    del buf_ref
    _mlp_body(cat_ref, edge_ref, w1st_ref, w1e_ref, b1_ref, w2_ref,
              b2_ref, w3_ref, b3_ref, out_ref)


def _edge_mlp_chunk(cat, edge_input, w1t, b1, w2t, b2, w3t, b3, buf, base_blk):
    ec = cat.shape[0]
    e = edge_input.shape[0]
    eb = 2000
    nblk = ec // eb
    d1 = w1t.shape[1]
    d2 = w2t.shape[1]
    d3 = w3t.shape[1]
    loc = lambda r, c: pl.BlockSpec((r, c), lambda i: (i, 0))
    glb = lambda r, c: pl.BlockSpec((r, c), lambda i: (i + base_blk, 0))
    full = lambda r, c: pl.BlockSpec((r, c), lambda i: (0, 0))
    in_specs = [
        loc(eb, 2 * H), glb(eb, H),
        full(2 * H, d1), full(H, d1), full(1, d1),
        full(d1, d2), full(1, d2),
        full(d2, d3), full(1, d3),
    ]
    args = [cat, edge_input,
            w1t[0:2 * H], w1t[2 * H:3 * H], b1[None, :],
            w2t, b2[None, :], w3t, b3[None, :]]
    kwargs = {}
    body = _mlp_body
    if buf is not None:
        in_specs.append(pl.BlockSpec(memory_space=pl.ANY))
        args.append(buf)
        body = _mlp_body_buf
        kwargs["input_output_aliases"] = {9: 0}
    return pl.pallas_call(
        body,
        grid=(nblk,),
        in_specs=in_specs,
        out_specs=pl.BlockSpec((eb, d3), lambda i: (i + base_blk, 0)),
        out_shape=jax.ShapeDtypeStruct((e, d3), jnp.float32),
        **kwargs,
    )(*args)


def kernel(node_attr, edge_input, edge_index, conv_w, conv_b,
           W1, b1, W2, b2, W3, b3):
    n = node_attr.shape[0]
    node_attr2d = node_attr.reshape(n, 4 * H)
    emb = _node_emb(node_attr2d, conv_w, conv_b)
    row = edge_index[0].astype(jnp.int32)
    col = edge_index[1].astype(jnp.int32)
    nch = 5
    e = row.shape[0]
    ec = e // nch
    w1t = W1.T
    cats = [
        _gather_src_tgt(emb, row, col, c * ec, ec)
        for c in range(nch)
    ]
    out = None
    eb = 2000
    for c in range(nch):
        out = _edge_mlp_chunk(cats[c], edge_input, w1t, b1, W2.T, b2, W3.T,
                              b3, out, c * (ec // eb))
    return out
